# baseline (speedup 1.0000x reference)
import functools

import jax
import jax.numpy as jnp
from jax import lax
from jax.experimental import pallas as pl
from jax.experimental.pallas import tpu as pltpu
from jax.experimental.pallas import tpu_sc as plsc

B, S = 4096, 200
TABLE_SIZE = 1000000
MULT = 2654435761

NC, NS = 2, 16
NW = NC * NS
TOTAL = B * S
ROWS_W = B // NW
RB = 16
NB = ROWS_W // RB
BLK = RB * S
PAD = 8
NCH = 13


MULT2 = (MULT * MULT) % (1 << 32)
MULT3 = (MULT * MULT * MULT) % (1 << 32)


def _row_hash_body(xb, i2, i3, i4):
    m1 = jnp.uint32(MULT)
    m2 = jnp.uint32(MULT2)
    m3 = jnp.uint32(MULT3)
    ts = jnp.uint32(TABLE_SIZE)
    lanes = lax.iota(jnp.uint32, 16)
    ge1 = lanes >= jnp.uint32(1)
    ge2 = lanes >= jnp.uint32(2)
    ge3 = lanes >= jnp.uint32(3)

    def body(r, carry):
        p0 = PAD + r * S
        q0 = r * S
        for c in range(NCH):
            s0 = 184 if c == NCH - 1 else 16 * c
            p = p0 + s0
            q = q0 + s0
            t = xb[pl.ds(p, 16)].astype(jnp.uint32)
            tm1 = xb[pl.ds(p - 1, 16)].astype(jnp.uint32)
            tm2 = xb[pl.ds(p - 2, 16)].astype(jnp.uint32)
            tm3 = xb[pl.ds(p - 3, 16)].astype(jnp.uint32)
            if c == 0:
                tm1 = jnp.where(ge1, tm1, jnp.uint32(0))
                tm2 = jnp.where(ge2, tm2, jnp.uint32(0))
                tm3 = jnp.where(ge3, tm3, jnp.uint32(0))
            h2 = tm1 * m1 + t
            h3 = tm2 * m2 + h2
            h4 = tm3 * m3 + h3
            i2[pl.ds(q, 16)] = (h2 % ts).astype(jnp.int32)
            i3[pl.ds(q, 16)] = (h3 % ts).astype(jnp.int32)
            i4[pl.ds(q, 16)] = (h4 % ts).astype(jnp.int32)
        return carry

    return body


def _fused_body(x_hbm, t2, t3, t4, out_hbm,
                xb0, xb1,
                i20, i30, i40, i21, i31, i41,
                v20, v30, v40, v21, v31, v41,
                sx0, sx1, sg0, sg1, ss0, ss1):
    wid = lax.axis_index("s") * NC + lax.axis_index("c")
    row0 = wid * ROWS_W
    tabs = [t2, t3, t4]
    xb = [xb0, xb1]
    idx = [[i20, i30, i40], [i21, i31, i41]]
    val = [[v20, v30, v40], [v21, v31, v41]]
    sx = [sx0, sx1]
    sg = [sg0, sg1]
    ss = [ss0, ss1]

    def x_src(j):
        return x_hbm.at[pl.ds((row0 + j * RB) * S, BLK)]

    def out_dst(j, n):
        return out_hbm.at[pl.ds(n * TOTAL + (row0 + j * RB) * S, BLK)]

    hx = [None] * NB
    hg = [None] * NB
    hs = [None] * NB
    hx[0] = pltpu.async_copy(x_src(0), xb[0].at[pl.ds(PAD, BLK)], sx[0])
    for j in range(NB):
        b = j % 2
        hx[j].wait()
        if j + 1 < NB:
            hx[j + 1] = pltpu.async_copy(
                x_src(j + 1), xb[1 - b].at[pl.ds(PAD, BLK)], sx[1 - b])
        body = _row_hash_body(xb[b], *idx[b])
        lax.fori_loop(0, RB, body, 0)
        if j >= 2:
            for h in hs[j - 2]:
                h.wait()
        hg[j] = [
            pltpu.async_copy(tabs[n].at[idx[b][n]], val[b][n], sg[b])
            for n in range(3)]
        if j >= 1:
            for h in hg[j - 1]:
                h.wait()
            hs[j - 1] = [
                pltpu.async_copy(val[1 - b][n], out_dst(j - 1, n), ss[1 - b])
                for n in range(3)]
    bl = (NB - 1) % 2
    for h in hg[NB - 1]:
        h.wait()
    hs[NB - 1] = [
        pltpu.async_copy(val[bl][n], out_dst(NB - 1, n), ss[bl])
        for n in range(3)]
    for h in hs[NB - 2]:
        h.wait()
    for h in hs[NB - 1]:
        h.wait()


@functools.cache
def _fused():
    return pl.kernel(
        _fused_body,
        out_type=jax.ShapeDtypeStruct((3 * TOTAL,), jnp.float32),
        mesh=plsc.VectorSubcoreMesh(core_axis_name="c", subcore_axis_name="s",
                                    num_cores=NC, num_subcores=NS),
        scratch_types=(
            [pltpu.VMEM((PAD + BLK,), jnp.int32) for _ in range(2)]
            + [pltpu.VMEM((BLK,), jnp.int32) for _ in range(6)]
            + [pltpu.VMEM((BLK,), jnp.float32) for _ in range(6)]
            + [pltpu.SemaphoreType.DMA for _ in range(6)]
        ),
    )


@jax.jit
def kernel(x, table_2, table_3, table_4):
    out = _fused()(x.reshape(-1), table_2, table_3, table_4)
    return out.reshape(3, B, S)
